# baseline (speedup 1.0000x reference)
import jax
import jax.numpy as jnp
from jax import lax
from jax.experimental import pallas as pl
from jax.experimental.pallas import tpu as pltpu
from jax.experimental.pallas import tpu_sc as plsc

D = 64
H = 128
B = 4096
L = 200
NC = 2
NS = 16
NW = NC * NS
BPW = B // NW
NCH = 2
CH = L // NCH
NV = D // 16

_mesh = plsc.VectorSubcoreMesh(
    core_axis_name="c", subcore_axis_name="s", num_cores=NC, num_subcores=NS)


DP = 128


def _sum_body(x_hbm, tab_hbm, out_hbm, idx_v, rows_v, out_v, sem):
    wid = lax.axis_index("s") * NC + lax.axis_index("c")
    base = wid * BPW
    pltpu.sync_copy(x_hbm.at[pl.ds(base, BPW)], idx_v)

    def row(r, carry):
        cps = [pltpu.async_copy(tab_hbm.at[idx_v.at[r, ch]], rows_v.at[ch], sem)
               for ch in range(NCH)]
        for cp in cps:
            cp.wait()
        accs = tuple(jnp.zeros((16,), jnp.float32) for _ in range(NV))
        for ch in range(NCH):
            def inner(j, acc):
                return tuple(acc[c] + rows_v[ch, j, pl.ds(c * 16, 16)]
                             for c in range(NV))
            accs = lax.fori_loop(0, CH, inner, accs)
        for c in range(NV):
            out_v[r, pl.ds(c * 16, 16)] = accs[c]
        return carry

    lax.fori_loop(0, BPW, row, 0)
    pltpu.sync_copy(out_v, out_hbm.at[pl.ds(base, BPW)])


_sum_kernel = pl.kernel(
    _sum_body,
    out_type=jax.ShapeDtypeStruct((B, D), jnp.float32),
    mesh=_mesh,
    scratch_types=[
        pltpu.VMEM((BPW, NCH, CH), jnp.int32),
        pltpu.VMEM((NCH, CH, DP), jnp.float32),
        pltpu.VMEM((BPW, D), jnp.float32),
        pltpu.SemaphoreType.DMA,
    ],
    compiler_params=pltpu.CompilerParams(use_tc_tiling_on_sc=True),
)


VOCAB_N = 1000000
NBLK = VOCAB_N // DP
TAILV = VOCAB_N - NBLK * DP
TPW2 = 123


def _tp_sc_body(tabT_hbm, out_hbm, in_v, tout_v, tin_v, sems):
    wid = lax.axis_index("s") * NC + lax.axis_index("c")
    iota = lax.iota(jnp.int32, 16)
    didx = [g * 16 + iota for g in range(NV)]

    def start_in(cb, buf):
        return pltpu.async_copy(
            tabT_hbm.at[:, pl.ds(cb * DP, DP)], in_v.at[buf], sems.at[buf])

    def transpose_out(cb, buf):
        def vrow(vl, carry):
            col = jnp.full((16,), vl, jnp.int32)
            for g in range(NV):
                tout_v[buf, vl, pl.ds(g * 16, 16)] = plsc.load_gather(
                    in_v.at[buf], [didx[g], col])
            return carry
        lax.fori_loop(0, DP, vrow, 0)
        pltpu.async_copy(
            tout_v.at[buf], out_hbm.at[pl.ds(cb * DP, DP)], sems.at[2 + buf])

    def wait_in(cb, buf):
        pltpu.make_async_copy(
            tabT_hbm.at[:, pl.ds(cb * DP, DP)], in_v.at[buf],
            sems.at[buf]).wait()

    def wait_out(cb, buf):
        pltpu.make_async_copy(
            tout_v.at[buf], out_hbm.at[pl.ds(cb * DP, DP)],
            sems.at[2 + buf]).wait()

    cb0 = wid

    @pl.when(cb0 < NBLK)
    def _():
        start_in(cb0, 0)

    def step(t2, carry):
        for ph in range(2):
            cb = wid + 32 * (2 * t2 + ph)
            nxt = wid + 32 * (2 * t2 + ph + 1)

            @pl.when(nxt < NBLK)
            def _():
                start_in(nxt, 1 - ph)

            @pl.when(cb < NBLK)
            def _():
                wait_in(cb, ph)

                @pl.when(t2 > 0)
                def _():
                    wait_out(cb - 64, ph)
                transpose_out(cb, ph)
        return carry

    lax.fori_loop(0, TPW2, step, 0)
    for ph in range(2):
        wait_out(0, ph)

    @pl.when(wid == 0)
    def _():
        pltpu.sync_copy(tabT_hbm.at[:, pl.ds(NBLK * DP, TAILV)], tin_v)

        def vrow(vl, carry):
            col = jnp.full((16,), vl, jnp.int32)
            for g in range(NV):
                tout_v[0, vl, pl.ds(g * 16, 16)] = plsc.load_gather(
                    tin_v, [didx[g], col])
            return carry
        lax.fori_loop(0, TAILV, vrow, 0)
        pltpu.sync_copy(tout_v.at[0, pl.ds(0, TAILV)],
                        out_hbm.at[pl.ds(NBLK * DP, TAILV)])


_tp_sc = pl.kernel(
    _tp_sc_body,
    out_type=jax.ShapeDtypeStruct((VOCAB_N, DP), jnp.float32),
    mesh=_mesh,
    scratch_types=[
        pltpu.VMEM((2, D, DP), jnp.float32),
        pltpu.VMEM((2, DP, DP), jnp.float32),
        pltpu.VMEM((D, TAILV), jnp.float32),
        pltpu.SemaphoreType.DMA((4,)),
    ],
    compiler_params=pltpu.CompilerParams(
        use_tc_tiling_on_sc=True, needs_layout_passes=False),
)


def _mlp_body(s_ref, w1_ref, b1_ref, w2_ref, b2_ref, o_ref):
    avg = s_ref[...] * (1.0 / L)
    h = jnp.dot(avg, w1_ref[...], preferred_element_type=jnp.float32)
    h = jnp.maximum(h + b1_ref[...], 0.0)
    o_ref[...] = jnp.dot(h, w2_ref[...],
                         preferred_element_type=jnp.float32) + b2_ref[...]


def _mlp(sums, W1, b1, W2, b2):
    return pl.pallas_call(
        _mlp_body,
        out_shape=jax.ShapeDtypeStruct((B, 2), jnp.float32),
    )(sums, W1, b1, W2, b2)


def kernel(x, table, W1, b1, W2, b2):
    x3 = x.astype(jnp.int32).reshape(B, NCH, CH)
    tab128 = _tp_sc(table.T)
    sums = _sum_kernel(x3, tab128)
    return _mlp(sums, W1, b1.reshape(1, H), W2, b2.reshape(1, 2))


# speedup vs baseline: 2.3606x; 2.3606x over previous
import jax
import jax.numpy as jnp
from jax import lax
from jax.experimental import pallas as pl
from jax.experimental.pallas import tpu as pltpu
from jax.experimental.pallas import tpu_sc as plsc

D = 64
H = 128
B = 4096
L = 200
NC = 2
NS = 16
NW = NC * NS
BPW = B // NW
NCH = 2
CH = L // NCH
NV = D // 16

_mesh = plsc.VectorSubcoreMesh(
    core_axis_name="c", subcore_axis_name="s", num_cores=NC, num_subcores=NS)


DP = 128


def _sum_body(x_hbm, tab_hbm, out_hbm, idx_v, rows_v, out_v, sem):
    wid = lax.axis_index("s") * NC + lax.axis_index("c")
    base = wid * BPW
    pltpu.sync_copy(x_hbm.at[pl.ds(base, BPW)], idx_v)

    def row(r, carry):
        cps = [pltpu.async_copy(tab_hbm.at[idx_v.at[r, ch]], rows_v.at[ch], sem)
               for ch in range(NCH)]
        for cp in cps:
            cp.wait()
        accs = tuple(jnp.zeros((16,), jnp.float32) for _ in range(NV))
        for ch in range(NCH):
            def inner(j4, acc):
                for u in range(4):
                    j = j4 * 4 + u
                    acc = tuple(acc[c] + rows_v[ch, j, pl.ds(c * 16, 16)]
                                for c in range(NV))
                return acc
            accs = lax.fori_loop(0, CH // 4, inner, accs)
        for c in range(NV):
            out_v[r, pl.ds(c * 16, 16)] = accs[c]
        return carry

    lax.fori_loop(0, BPW, row, 0)
    pltpu.sync_copy(out_v, out_hbm.at[pl.ds(base, BPW)])


_sum_kernel = pl.kernel(
    _sum_body,
    out_type=jax.ShapeDtypeStruct((B, D), jnp.float32),
    mesh=_mesh,
    scratch_types=[
        pltpu.VMEM((BPW, NCH, CH), jnp.int32),
        pltpu.VMEM((NCH, CH, DP), jnp.float32),
        pltpu.VMEM((BPW, D), jnp.float32),
        pltpu.SemaphoreType.DMA,
    ],
    compiler_params=pltpu.CompilerParams(use_tc_tiling_on_sc=True),
)


VOCAB_N = 1000000
TK = 2048


def _tp_body(t_ref, o_ref):
    tt = jnp.swapaxes(t_ref[...], 0, 1)
    o_ref[...] = jnp.concatenate(
        [tt, jnp.zeros((TK, DP - D), jnp.float32)], axis=1)


def _transpose_pad(tableT):
    return pl.pallas_call(
        _tp_body,
        grid=(pl.cdiv(VOCAB_N, TK),),
        in_specs=[pl.BlockSpec((D, TK), lambda i: (0, i))],
        out_specs=pl.BlockSpec((TK, DP), lambda i: (i, 0)),
        out_shape=jax.ShapeDtypeStruct((VOCAB_N, DP), jnp.float32),
    )(tableT)


def _mlp_body(s_ref, w1_ref, b1_ref, w2_ref, b2_ref, o_ref):
    avg = s_ref[...] * (1.0 / L)
    h = jnp.dot(avg, w1_ref[...], preferred_element_type=jnp.float32)
    h = jnp.maximum(h + b1_ref[...], 0.0)
    o_ref[...] = jnp.dot(h, w2_ref[...],
                         preferred_element_type=jnp.float32) + b2_ref[...]


def _mlp(sums, W1, b1, W2, b2):
    return pl.pallas_call(
        _mlp_body,
        out_shape=jax.ShapeDtypeStruct((B, 2), jnp.float32),
    )(sums, W1, b1, W2, b2)


def kernel(x, table, W1, b1, W2, b2):
    x3 = x.astype(jnp.int32).reshape(B, NCH, CH)
    tab128 = _transpose_pad(table.T)
    sums = _sum_kernel(x3, tab128)
    return _mlp(sums, W1, b1.reshape(1, H), W2, b2.reshape(1, 2))
